# baseline (speedup 1.0000x reference)
import jax
import jax.numpy as jnp
import numpy as np
from jax.experimental import pallas as pl
from jax.experimental.pallas import tpu as pltpu

IOU_THR = 0.45
K_NMS = 1000
KP = 1024
NROW = 16


def _nms_kernel(drow_ref, dcol_ref, out_ref):
    d = drow_ref[0]
    dc = dcol_ref[0]
    x1r, y1r, x2r, y2r = d[1:2], d[2:3], d[3:4], d[4:5]
    x1c, y1c, x2c, y2c = dc[:, 1:2], dc[:, 2:3], dc[:, 3:4], dc[:, 4:5]
    area_r = jnp.maximum(x2r - x1r, 0.0) * jnp.maximum(y2r - y1r, 0.0)
    area_c = jnp.maximum(x2c - x1c, 0.0) * jnp.maximum(y2c - y1c, 0.0)
    xx1 = jnp.maximum(x1c, x1r)
    yy1 = jnp.maximum(y1c, y1r)
    xx2 = jnp.minimum(x2c, x2r)
    yy2 = jnp.minimum(y2c, y2r)
    inter = jnp.maximum(xx2 - xx1, 0.0) * jnp.maximum(yy2 - yy1, 0.0)
    iou = inter / (area_c + area_r - inter + 1e-9)
    jidx = jax.lax.broadcasted_iota(jnp.int32, (KP, KP), 0)
    iidx = jax.lax.broadcasted_iota(jnp.int32, (KP, KP), 1)
    S = jnp.where((iou > IOU_THR) & (jidx < iidx), 1.0, 0.0)

    def cond(c):
        _, changed = c
        return changed

    def body(c):
        keep, _ = c
        sup = jnp.dot(keep, S, preferred_element_type=jnp.float32)
        newkeep = jnp.where(sup > 0.5, 0.0, 1.0)
        return newkeep, jnp.any(newkeep != keep)

    keep0 = jnp.ones((1, KP), jnp.float32)
    keep, _ = jax.lax.while_loop(cond, body, (keep0, jnp.bool_(True)))
    out_ref[0] = d * keep


def _nms_pallas(drow, dcol):
    b = drow.shape[0]
    return pl.pallas_call(
        _nms_kernel,
        grid=(b,),
        in_specs=[
            pl.BlockSpec((1, NROW, KP), lambda i: (i, 0, 0)),
            pl.BlockSpec((1, KP, NROW), lambda i: (i, 0, 0)),
        ],
        out_specs=pl.BlockSpec((1, NROW, KP), lambda i: (i, 0, 0)),
        out_shape=jax.ShapeDtypeStruct((b, NROW, KP), jnp.float32),
    )(drow, dcol)


def _conv(x, w, stride=1):
    return jax.lax.conv_general_dilated(
        x, w, (stride, stride), 'SAME',
        dimension_numbers=('NCHW', 'OIHW', 'NCHW'))


def _up2(x):
    return jnp.repeat(jnp.repeat(x, 2, axis=2), 2, axis=3)


def _anchor_centers(h, w, stride, na):
    sy, sx = jnp.meshgrid(jnp.arange(h), jnp.arange(w), indexing='ij')
    ac = (jnp.stack([sx, sy], -1).astype(jnp.float32) * stride).reshape(-1, 2)
    return jnp.repeat(ac[:, None, :], na, axis=1).reshape(-1, 2)


def kernel(x, stem1_w, stem2_w, c3_w, c4_w, c5_w, lat3_w, lat4_w, lat5_w,
           smooth3_w, smooth4_w, smooth5_w, head_w, cls_w, box_w, kps_w):
    relu = jax.nn.relu
    h = relu(_conv(x, stem1_w, 2))
    h = relu(_conv(h, stem2_w, 2))
    c3 = relu(_conv(h, c3_w, 2))
    c4 = relu(_conv(c3, c4_w, 2))
    c5 = relu(_conv(c4, c5_w, 2))
    p5 = _conv(c5, lat5_w)
    p4 = _conv(c4, lat4_w) + _up2(p5)
    p3 = _conv(c3, lat3_w) + _up2(p4)
    p3 = relu(_conv(p3, smooth3_w))
    p4 = relu(_conv(p4, smooth4_w))
    p5 = relu(_conv(p5, smooth5_w))
    na, nc = 2, 1
    sc_l, bx_l, kp_l = [], [], []
    for f, s in zip((p3, p4, p5), (8, 16, 32)):
        t = relu(_conv(f, head_w))
        cls = _conv(t, cls_w)
        box = _conv(t, box_w)
        kp = _conv(t, kps_w)
        b, _, hh, ww = cls.shape
        ac = _anchor_centers(hh, ww, s, na)
        sm = jax.nn.sigmoid(cls.reshape(b, na, nc, hh, ww)
                            .transpose(0, 3, 4, 1, 2).reshape(b, -1, nc))
        sc = jnp.max(sm, axis=-1)
        bb = box.reshape(b, na, 4, hh, ww).transpose(0, 3, 4, 1, 2).reshape(b, -1, 4) * s
        x1 = ac[None, :, 0] - bb[..., 0]
        y1 = ac[None, :, 1] - bb[..., 1]
        x2 = ac[None, :, 0] + bb[..., 2]
        y2 = ac[None, :, 1] + bb[..., 3]
        bxs = jnp.stack([x1, y1, x2, y2], -1)
        kk = kp.reshape(b, na, 10, hh, ww).transpose(0, 3, 4, 1, 2).reshape(b, -1, 10) * s
        kx = ac[None, :, 0:1] + kk[..., 0::2]
        ky = ac[None, :, 1:2] + kk[..., 1::2]
        kps_dec = jnp.stack([kx, ky], -1).reshape(b, -1, 10)
        sc_l.append(sc)
        bx_l.append(bxs)
        kp_l.append(kps_dec)
    scores = jnp.concatenate(sc_l, 1)
    boxes = jnp.concatenate(bx_l, 1)
    kpss = jnp.concatenate(kp_l, 1)
    NPAD = 16896
    sc_pad = jnp.pad(scores, ((0, 0), (0, NPAD - scores.shape[1])), constant_values=-1.0)
    iota = jnp.broadcast_to(jnp.arange(NPAD, dtype=jnp.int32)[None], (scores.shape[0], NPAD))
    neg_sorted, idx_sorted = jax.lax.sort_key_val(-sc_pad, iota, dimension=1)
    vals = -neg_sorted[:, :K_NMS]
    idx = idx_sorted[:, :K_NMS]
    boxes_k = jnp.take_along_axis(boxes, idx[..., None], axis=1)
    kps_k = jnp.take_along_axis(kpss, idx[..., None], axis=1)

    b = vals.shape[0]
    fields = jnp.concatenate([vals[..., None], boxes_k, kps_k,
                              jnp.zeros((b, K_NMS, 1), jnp.float32)], -1)
    dcol = jnp.pad(fields, ((0, 0), (0, KP - K_NMS), (0, 0)))
    drow = dcol.transpose(0, 2, 1)
    res = _nms_pallas(drow, dcol)
    return res[:, :15, :K_NMS].transpose(0, 2, 1)


# speedup vs baseline: 1.0085x; 1.0085x over previous
import jax
import jax.numpy as jnp
import numpy as np
from jax.experimental import pallas as pl
from jax.experimental.pallas import tpu as pltpu

IOU_THR = 0.45
K_NMS = 1000
KP = 1024
NROW = 16
NPAD = 16896
ROWS = 132
OUTB = 2080
NB = 4


def _select_kernel(s_ref, p_ref):
    u = jax.lax.bitcast_convert_type(s_ref[...], jnp.int32)

    def count_ge(t):
        c = jnp.where(u >= t, 1.0, 0.0)
        return jnp.sum(jnp.sum(c, axis=2), axis=1).reshape(NB, 1, 1)

    def body(_, carry):
        lo, hi = carry
        mid = lo + ((hi - lo) >> 1)
        c = count_ge(mid)
        big = c >= float(K_NMS)
        return (jnp.where(big, mid, lo), jnp.where(big, hi, mid))

    lo0 = jnp.zeros((NB, 1, 1), jnp.int32)
    hi0 = jnp.full((NB, 1, 1), 0x3F800001, jnp.int32)
    v, _ = jax.lax.fori_loop(0, 31, body, (lo0, hi0))

    cnt_gt = jnp.sum(jnp.sum(jnp.where(u > v, 1.0, 0.0), axis=2), axis=1)
    m = float(K_NMS) - cnt_gt.reshape(NB, 1, 1)

    l_iota = jax.lax.broadcasted_iota(jnp.int32, (128, 128), 0)
    l_iota_t = jax.lax.broadcasted_iota(jnp.int32, (128, 128), 1)
    su128 = jnp.where(l_iota < l_iota_t, 1.0, 0.0)
    r_iota = jax.lax.broadcasted_iota(jnp.int32, (ROWS, ROWS), 0)
    r_iota_t = jax.lax.broadcasted_iota(jnp.int32, (ROWS, ROWS), 1)
    sl132 = jnp.where(r_iota_t < r_iota, 1.0, 0.0)
    ones_col = jnp.ones((128, 1), jnp.float32)

    def excl_prefix(mask):
        in_row = jnp.dot(mask, su128, preferred_element_type=jnp.float32)
        row_tot = jnp.dot(mask, ones_col, preferred_element_type=jnp.float32)
        prev_rows = jnp.dot(sl132, row_tot, preferred_element_type=jnp.float32)
        return in_row + prev_rows

    ridx = jax.lax.broadcasted_iota(jnp.int32, (ROWS, 128), 0)
    lidx = jax.lax.broadcasted_iota(jnp.int32, (ROWS, 128), 1)
    flat_i = ridx * 128 + lidx
    dump = K_NMS + (flat_i >> 4)

    for b in range(NB):
        ub = u[b]
        vb = v[b]
        eq = jnp.where(ub == vb, 1.0, 0.0)
        gtm = jnp.where(ub > vb, 1.0, 0.0)
        tie_pos = excl_prefix(eq)
        member = gtm + eq * jnp.where(tie_pos < m[b], 1.0, 0.0)
        pos = excl_prefix(member)
        p_local = jnp.where(member > 0.5, pos.astype(jnp.int32), dump)
        p_ref[b] = b * OUTB + p_local


def _select_pallas(scores3):
    return pl.pallas_call(
        _select_kernel,
        out_shape=jax.ShapeDtypeStruct((NB, ROWS, 128), jnp.int32),
    )(scores3)


def _nms_direct_kernel(vals_ref, box_ref, kps_ref, out_ref):
    KN = K_NMS
    v = vals_ref[0]
    bx = box_ref[0]
    kp = kps_ref[0]
    bt = jnp.transpose(bx, (1, 0))
    x1c, y1c, x2c, y2c = bx[:, 0:1], bx[:, 1:2], bx[:, 2:3], bx[:, 3:4]
    x1r, y1r, x2r, y2r = bt[0:1], bt[1:2], bt[2:3], bt[3:4]
    area_r = jnp.maximum(x2r - x1r, 0.0) * jnp.maximum(y2r - y1r, 0.0)
    area_c = jnp.maximum(x2c - x1c, 0.0) * jnp.maximum(y2c - y1c, 0.0)
    xx1 = jnp.maximum(x1c, x1r)
    yy1 = jnp.maximum(y1c, y1r)
    xx2 = jnp.minimum(x2c, x2r)
    yy2 = jnp.minimum(y2c, y2r)
    inter = jnp.maximum(xx2 - xx1, 0.0) * jnp.maximum(yy2 - yy1, 0.0)
    iou = inter / (area_c + area_r - inter + 1e-9)
    jidx = jax.lax.broadcasted_iota(jnp.int32, (KN, KN), 0)
    iidx = jax.lax.broadcasted_iota(jnp.int32, (KN, KN), 1)
    supmat = jnp.where((iou > IOU_THR) & (jidx < iidx), 1.0, 0.0)

    def cond(c):
        return c[1]

    def body(c):
        keep, _ = c
        sup = jnp.dot(keep, supmat, preferred_element_type=jnp.float32)
        nk = jnp.where(sup > 0.5, 0.0, 1.0)
        return nk, jnp.any(nk != keep)

    keep, _ = jax.lax.while_loop(
        cond, body, (jnp.ones((1, KN), jnp.float32), jnp.bool_(True)))
    planar = jnp.concatenate([v, bt, jnp.transpose(kp, (1, 0))], axis=0)
    out_ref[0] = planar * keep


def _nms_direct_pallas(vals, boxes_k, kps_k):
    return pl.pallas_call(
        _nms_direct_kernel,
        grid=(NB,),
        in_specs=[
            pl.BlockSpec((1, 1, K_NMS), lambda i: (i, 0, 0)),
            pl.BlockSpec((1, K_NMS, 4), lambda i: (i, 0, 0)),
            pl.BlockSpec((1, K_NMS, 10), lambda i: (i, 0, 0)),
        ],
        out_specs=pl.BlockSpec((1, 15, K_NMS), lambda i: (i, 0, 0)),
        out_shape=jax.ShapeDtypeStruct((NB, 15, K_NMS), jnp.float32),
    )(vals.reshape(NB, 1, K_NMS), boxes_k, kps_k)


def _sort_nms_kernel(fc_ref, out_ref):
    fc = fc_ref[0]
    iota_s = jax.lax.broadcasted_iota(jnp.int32, (KP, KP), 0)
    iota_l = jax.lax.broadcasted_iota(jnp.int32, (KP, KP), 1)
    planar = jnp.transpose(fc, (1, 0))

    vcol = iota_s[:, :1] < K_NMS
    vrow = iota_l[:1, :] < K_NMS
    s_col = jnp.where(vcol, fc[:, 0:1], -1.0)
    s_row = jnp.where(vrow, planar[0:1, :], -1.0)
    ix_col = jnp.where(vcol, fc[:, 15:16], 1e9)
    ix_row = jnp.where(vrow, planar[15:16, :], 1e9)

    gt = (s_col > s_row) | ((s_col == s_row) & (ix_col < ix_row))
    lt = (s_row > s_col) | ((s_row == s_col) & (ix_row < ix_col))
    rank_col = jnp.sum(jnp.where(lt, 1.0, 0.0), axis=1, keepdims=True)

    q = jnp.where(rank_col == iota_l.astype(jnp.float32), 1.0, 0.0)
    sorted_cols = jax.lax.dot_general(q, fc, (((0,), (0,)), ((), ())),
                                      preferred_element_type=jnp.float32,
                                      precision=jax.lax.Precision.HIGHEST)
    sorted_planar = jnp.transpose(sorted_cols, (1, 0))

    mrow = jnp.where(vrow, 1.0, 0.0)
    mcol = jnp.where(vcol, 1.0, 0.0)
    x1r, y1r = sorted_planar[1:2] * mrow, sorted_planar[2:3] * mrow
    x2r, y2r = sorted_planar[3:4] * mrow, sorted_planar[4:5] * mrow
    x1c, y1c = sorted_cols[:, 1:2] * mcol, sorted_cols[:, 2:3] * mcol
    x2c, y2c = sorted_cols[:, 3:4] * mcol, sorted_cols[:, 4:5] * mcol
    area_r = jnp.maximum(x2r - x1r, 0.0) * jnp.maximum(y2r - y1r, 0.0)
    area_c = jnp.maximum(x2c - x1c, 0.0) * jnp.maximum(y2c - y1c, 0.0)
    xx1 = jnp.maximum(x1c, x1r)
    yy1 = jnp.maximum(y1c, y1r)
    xx2 = jnp.minimum(x2c, x2r)
    yy2 = jnp.minimum(y2c, y2r)
    inter = jnp.maximum(xx2 - xx1, 0.0) * jnp.maximum(yy2 - yy1, 0.0)
    iou = inter / (area_c + area_r - inter + 1e-9)
    supmat = jnp.where((iou > IOU_THR) & (iota_s < iota_l), 1.0, 0.0)

    def cond(c):
        _, changed = c
        return changed

    def body(c):
        keep, _ = c
        sup = jnp.dot(keep, supmat, preferred_element_type=jnp.float32)
        newkeep = jnp.where(sup > 0.5, 0.0, 1.0)
        return newkeep, jnp.any(newkeep != keep)

    keep0 = jnp.ones((1, KP), jnp.float32)
    keep, _ = jax.lax.while_loop(cond, body, (keep0, jnp.bool_(True)))
    out_ref[0] = sorted_planar * keep


def _sort_nms_pallas(fc):
    return pl.pallas_call(
        _sort_nms_kernel,
        grid=(NB,),
        in_specs=[pl.BlockSpec((1, KP, NROW), lambda i: (i, 0, 0))],
        out_specs=pl.BlockSpec((1, NROW, KP), lambda i: (i, 0, 0)),
        out_shape=jax.ShapeDtypeStruct((NB, NROW, KP), jnp.float32),
    )(fc)


def _scatter_standin(fields_flat, p_flat):
    out = jnp.zeros((NB * OUTB, NROW), jnp.float32)
    return out.at[p_flat].set(fields_flat, mode='drop')


def _conv(x, w, stride=1):
    return jax.lax.conv_general_dilated(
        x, w, (stride, stride), 'SAME',
        dimension_numbers=('NCHW', 'OIHW', 'NCHW'))


def _up2(x):
    return jnp.repeat(jnp.repeat(x, 2, axis=2), 2, axis=3)


def _anchor_centers(h, w, stride, na):
    sy, sx = jnp.meshgrid(jnp.arange(h), jnp.arange(w), indexing='ij')
    ac = (jnp.stack([sx, sy], -1).astype(jnp.float32) * stride).reshape(-1, 2)
    return jnp.repeat(ac[:, None, :], na, axis=1).reshape(-1, 2)


def kernel(x, stem1_w, stem2_w, c3_w, c4_w, c5_w, lat3_w, lat4_w, lat5_w,
           smooth3_w, smooth4_w, smooth5_w, head_w, cls_w, box_w, kps_w):
    relu = jax.nn.relu
    h = relu(_conv(x, stem1_w, 2))
    h = relu(_conv(h, stem2_w, 2))
    c3 = relu(_conv(h, c3_w, 2))
    c4 = relu(_conv(c3, c4_w, 2))
    c5 = relu(_conv(c4, c5_w, 2))
    p5 = _conv(c5, lat5_w)
    p4 = _conv(c4, lat4_w) + _up2(p5)
    p3 = _conv(c3, lat3_w) + _up2(p4)
    p3 = relu(_conv(p3, smooth3_w))
    p4 = relu(_conv(p4, smooth4_w))
    p5 = relu(_conv(p5, smooth5_w))
    na, nc = 2, 1
    sc_l, bx_l, kp_l = [], [], []
    for f, s in zip((p3, p4, p5), (8, 16, 32)):
        t = relu(_conv(f, head_w))
        cls = _conv(t, cls_w)
        box = _conv(t, box_w)
        kp = _conv(t, kps_w)
        b, _, hh, ww = cls.shape
        ac = _anchor_centers(hh, ww, s, na)
        sm = jax.nn.sigmoid(cls.reshape(b, na, nc, hh, ww)
                            .transpose(0, 3, 4, 1, 2).reshape(b, -1, nc))
        sc = jnp.max(sm, axis=-1)
        bb = box.reshape(b, na, 4, hh, ww).transpose(0, 3, 4, 1, 2).reshape(b, -1, 4) * s
        x1 = ac[None, :, 0] - bb[..., 0]
        y1 = ac[None, :, 1] - bb[..., 1]
        x2 = ac[None, :, 0] + bb[..., 2]
        y2 = ac[None, :, 1] + bb[..., 3]
        bxs = jnp.stack([x1, y1, x2, y2], -1)
        kk = kp.reshape(b, na, 10, hh, ww).transpose(0, 3, 4, 1, 2).reshape(b, -1, 10) * s
        kx = ac[None, :, 0:1] + kk[..., 0::2]
        ky = ac[None, :, 1:2] + kk[..., 1::2]
        kps_dec = jnp.stack([kx, ky], -1).reshape(b, -1, 10)
        sc_l.append(sc)
        bx_l.append(bxs)
        kp_l.append(kps_dec)
    scores = jnp.concatenate(sc_l, 1)
    boxes = jnp.concatenate(bx_l, 1)
    kpss = jnp.concatenate(kp_l, 1)

    vals, idx = jax.lax.top_k(scores, K_NMS)
    boxes_k = jnp.take_along_axis(boxes, idx[..., None], axis=1)
    kps_k = jnp.take_along_axis(kpss, idx[..., None], axis=1)

    res = _nms_direct_pallas(vals, boxes_k, kps_k)
    return res.transpose(0, 2, 1)
